# Initial kernel scaffold; baseline (speedup 1.0000x reference)
import jax
import jax.numpy as jnp
from jax.experimental import pallas as pl


def kernel(x, temperature, platt_a, platt_b, bin_correct, bin_total):
    raise NotImplementedError("write your pallas kernel here")


# speedup vs baseline: 1.0174x; 1.0174x over previous
import jax
import jax.numpy as jnp
from jax.experimental import pallas as pl
from jax.experimental.pallas import tpu as pltpu

_N_BINS = 15


def _stats_kernel(temp_ref, bc_ref, bt_ref, ece_ref, tout_ref, acc_ref):
    bc = bc_ref[...]
    bt = bt_ref[...]
    acc = bc / (bt + 1e-8)
    acc_ref[...] = acc
    lane_i = jax.lax.broadcasted_iota(jnp.int32, (1, 128), 1)
    mask = lane_i < _N_BINS
    lane = lane_i.astype(jnp.float32)
    conf = lane * (1.0 / (_N_BINS - 1)) + (0.5 / _N_BINS)
    n = jnp.sum(jnp.where(mask, bt, 0.0))
    contrib = jnp.where(mask, bt * jnp.abs(acc - conf), 0.0)
    ece = jnp.where(n > 0.0, jnp.sum(contrib) / jnp.maximum(n, 1e-8), 0.0)
    ece_ref[0, 0] = ece
    tout_ref[0, 0] = jnp.clip(temp_ref[0, 0], 0.1, 10.0)


def _stats(temperature, bin_correct, bin_total):
    bc = jnp.zeros((1, 128), jnp.float32).at[0, :_N_BINS].set(bin_correct)
    bt = jnp.zeros((1, 128), jnp.float32).at[0, :_N_BINS].set(bin_total)
    t2 = temperature.reshape(1, 1)
    ece, temp, acc = pl.pallas_call(
        _stats_kernel,
        out_shape=(
            jax.ShapeDtypeStruct((1, 1), jnp.float32),
            jax.ShapeDtypeStruct((1, 1), jnp.float32),
            jax.ShapeDtypeStruct((1, 128), jnp.float32),
        ),
        in_specs=[
            pl.BlockSpec(memory_space=pltpu.SMEM),
            pl.BlockSpec(memory_space=pltpu.VMEM),
            pl.BlockSpec(memory_space=pltpu.VMEM),
        ],
        out_specs=(
            pl.BlockSpec(memory_space=pltpu.SMEM),
            pl.BlockSpec(memory_space=pltpu.SMEM),
            pl.BlockSpec(memory_space=pltpu.VMEM),
        ),
    )(t2, bc, bt)
    return ece.reshape(()), temp.reshape(()), acc[0, :_N_BINS]


def kernel(x, temperature, platt_a, platt_b, bin_correct, bin_total):
    ece, temp, acc = _stats(temperature, bin_correct, bin_total)
    return (x, ece, temp, acc)
